# baseline (speedup 1.0000x reference)
import functools
import jax
import jax.numpy as jnp
from jax import lax
from jax.experimental import pallas as pl
from jax.experimental.pallas import tpu as pltpu
from jax.experimental.pallas import tpu_sc as plsc

_NC = 2
_NS = 16
_NW = _NC * _NS
_NIG = _NW // 2
_L = 16
_HP = 33


def _body(chunk, nchunks, b_per_ig, H, tpad_hbm, idx_hbm, out_hbm,
          ttile, idx0, idx1, stag0, stag1,
          sem_t, sem_i0, sem_i1, sem_w0, sem_w1):
    wid = lax.axis_index("s") * _NC + lax.axis_index("c")
    ig = wid // 2
    h = wid % 2
    base = ig * b_per_ig
    idxb = (idx0, idx1)
    stag = (stag0, stag1)
    sem_i = (sem_i0, sem_i1)
    sem_w = (sem_w0, sem_w1)

    pltpu.async_copy(tpad_hbm.at[h], ttile, sem_t).wait()

    def start_idx(c, b):
        pltpu.async_copy(idx_hbm.at[pl.ds(base + c * chunk, chunk)],
                         idxb[b], sem_i[b])

    def wait_idx(c, b):
        pltpu.make_async_copy(idx_hbm.at[pl.ds(base + c * chunk, chunk)],
                              idxb[b], sem_i[b]).wait()

    def start_write(c, b):
        pltpu.async_copy(
            stag[b].at[:, pl.ds(0, H)],
            out_hbm.at[pl.ds(base + c * chunk, chunk), pl.ds(h * H, H)],
            sem_w[b])

    def wait_write(c, b):
        pltpu.make_async_copy(
            stag[b].at[:, pl.ds(0, H)],
            out_hbm.at[pl.ds(base + c * chunk, chunk), pl.ds(h * H, H)],
            sem_w[b]).wait()

    lane = lax.iota(jnp.int32, _L)
    cols = [jnp.full((_L,), d, jnp.int32) for d in range(H)]

    def compute(b):
        @plsc.parallel_loop(0, chunk // _L, unroll=2)
        def group(r):
            iv = idxb[b][pl.ds(r * _L, _L)]
            rv = lane + r * _L
            for d0 in range(0, H, 8):
                vals = [plsc.load_gather(ttile, [iv, cols[d0 + k]])
                        for k in range(8)]
                for k in range(8):
                    plsc.store_scatter(stag[b], [rv, cols[d0 + k]], vals[k])

    for b in range(2):
        start_idx(b, b)

    def pair(g, carry):
        for b in range(2):
            c = g * 2 + b
            wait_idx(c, b)
            compute(b)
            start_write(c, b)
            wait_write(c, b)
            start_idx(c + 2, b)
        return carry

    npairs = nchunks // 2
    lax.fori_loop(0, npairs - 1, pair, 0)

    for b in range(2):
        c = (npairs - 1) * 2 + b
        wait_idx(c, b)
        compute(b)
        start_write(c, b)
    for b in range(2):
        c = (npairs - 1) * 2 + b
        wait_write(c, b)


def kernel(input_pos_tensors, table):
    B0, T = input_pos_tensors.shape
    V, D = table.shape
    B = B0 * T
    idx = input_pos_tensors.reshape(B).astype(jnp.int32)
    H = D // 2
    tpad = jnp.pad(table.reshape(V, 2, H), ((0, 0), (0, 0), (0, _HP - H)))
    tpad = jnp.moveaxis(tpad, 1, 0)

    b_per_ig = B // _NIG
    chunk = 512
    nchunks = b_per_ig // chunk

    mesh = plsc.VectorSubcoreMesh(
        core_axis_name="c", subcore_axis_name="s",
        num_cores=_NC, num_subcores=_NS)
    run = pl.kernel(
        functools.partial(_body, chunk, nchunks, b_per_ig, H),
        out_type=jax.ShapeDtypeStruct((B, D), jnp.float32),
        mesh=mesh,
        scratch_types=[
            pltpu.VMEM((V, _HP), jnp.float32),
            pltpu.VMEM((chunk,), jnp.int32),
            pltpu.VMEM((chunk,), jnp.int32),
            pltpu.VMEM((chunk, _HP), jnp.float32),
            pltpu.VMEM((chunk, _HP), jnp.float32),
            pltpu.SemaphoreType.DMA,
            pltpu.SemaphoreType.DMA,
            pltpu.SemaphoreType.DMA,
            pltpu.SemaphoreType.DMA,
            pltpu.SemaphoreType.DMA,
        ],
        compiler_params=pltpu.CompilerParams(
            use_tc_tiling_on_sc=False, needs_layout_passes=False,
            disable_bounds_checks=True),
    )
    out = run(tpad, idx)
    return out.reshape(B0, T, D)


# speedup vs baseline: 1.8933x; 1.8933x over previous
import functools
import jax
import jax.numpy as jnp
from jax import lax
from jax.experimental import pallas as pl
from jax.experimental.pallas import tpu as pltpu
from jax.experimental.pallas import tpu_sc as plsc

_NC = 2
_NS = 16
_NW = _NC * _NS
_NIG = _NW // 2
_L = 16
_CS = 288
_CC = 224
_NSC = 50
_NCC = 100


def _body(H, table_hbm, idx_hbm, out_hbm,
          table_sp, thalf, sidx0, sidx1, sbuf0, sbuf1,
          cidx0, cidx1, cstag0, cstag1,
          sem_th, sem_si0, sem_si1, sem_sg0, sem_sg1, sem_sw0, sem_sw1,
          sem_ci0, sem_ci1, sem_cw0, sem_cw1):
    wid = lax.axis_index("s") * _NC + lax.axis_index("c")
    ig = wid // 2
    h = wid % 2
    sbase = wid * (_CS * _NSC)
    cbase = _NW * (_CS * _NSC) + ig * (_CC * _NCC)
    sidx = (sidx0, sidx1)
    sbuf = (sbuf0, sbuf1)
    cidx = (cidx0, cidx1)
    cstag = (cstag0, cstag1)
    sem_si = (sem_si0, sem_si1)
    sem_sg = (sem_sg0, sem_sg1)
    sem_sw = (sem_sw0, sem_sw1)
    sem_ci = (sem_ci0, sem_ci1)
    sem_cw = (sem_cw0, sem_cw1)

    pltpu.async_copy(table_hbm.at[:, pl.ds(h * H, H)], thalf, sem_th)

    @pl.when(lax.axis_index("s") == 0)
    def _stage():
        pltpu.sync_copy(table_hbm, table_sp)
    plsc.subcore_barrier()
    pltpu.make_async_copy(table_hbm.at[:, pl.ds(h * H, H)], thalf,
                          sem_th).wait()

    def si_start(c, b):
        pltpu.async_copy(idx_hbm.at[pl.ds(sbase + c * _CS, _CS)],
                         sidx[b], sem_si[b])

    def si_wait(c, b):
        pltpu.make_async_copy(idx_hbm.at[pl.ds(sbase + c * _CS, _CS)],
                              sidx[b], sem_si[b]).wait()

    def sg_start(c, b):
        pltpu.async_copy(table_sp.at[sidx[b]], sbuf[b], sem_sg[b])

    def sg_wait(c, b):
        pltpu.make_async_copy(table_sp.at[sidx[b]], sbuf[b], sem_sg[b]).wait()

    def sw_start(c, b):
        pltpu.async_copy(sbuf[b], out_hbm.at[pl.ds(sbase + c * _CS, _CS)],
                         sem_sw[b])

    def sw_wait(c, b):
        pltpu.make_async_copy(sbuf[b],
                              out_hbm.at[pl.ds(sbase + c * _CS, _CS)],
                              sem_sw[b]).wait()

    def ci_start(q, j):
        pltpu.async_copy(idx_hbm.at[pl.ds(cbase + q * _CC, _CC)],
                         cidx[j], sem_ci[j])

    def ci_wait(q, j):
        pltpu.make_async_copy(idx_hbm.at[pl.ds(cbase + q * _CC, _CC)],
                              cidx[j], sem_ci[j]).wait()

    def cw_start(q, j):
        pltpu.async_copy(
            cstag[j], out_hbm.at[pl.ds(cbase + q * _CC, _CC),
                                 pl.ds(h * H, H)], sem_cw[j])

    def cw_wait(q, j):
        pltpu.make_async_copy(
            cstag[j], out_hbm.at[pl.ds(cbase + q * _CC, _CC),
                                 pl.ds(h * H, H)], sem_cw[j]).wait()

    def compute(j):
        @plsc.parallel_loop(0, _CC // _L, unroll=2)
        def group(r):
            iv = cidx[j][pl.ds(r * _L, _L)]
            ss = [iv[l] for l in range(_L)]
            vals = [[thalf[ss[l], pl.ds(d, _L)] for d in range(0, H, _L)]
                    for l in range(_L)]
            for l in range(_L):
                for k, d in enumerate(range(0, H, _L)):
                    cstag[j][r * _L + l, pl.ds(d, _L)] = vals[l][k]

    def copy_step(q, j, wait_prev, issue_next):
        ci_wait(q, j)
        if wait_prev:
            cw_wait(q - 2, j)
        compute(j)
        cw_start(q, j)
        if issue_next:
            ci_start(q + 2, j)

    for b in range(2):
        si_start(b, b)
        ci_start(b, b)
    si_wait(0, 0)
    sg_start(0, 0)

    sg_wait(0, 0)
    sw_start(0, 0)
    si_wait(1, 1)
    sg_start(1, 1)
    si_start(2, 0)
    copy_step(0, 0, False, True)
    copy_step(1, 1, False, True)
    sg_wait(1, 1)
    sw_start(1, 1)
    si_wait(2, 0)
    sw_wait(0, 0)
    sg_start(2, 0)
    si_start(3, 1)
    copy_step(2, 0, True, True)
    copy_step(3, 1, True, True)

    def pair(m, carry):
        for b in range(2):
            g = 2 * m + b
            sg_wait(g, b)
            sw_start(g, b)
            si_wait(g + 1, 1 - b)
            sw_wait(g - 1, 1 - b)
            sg_start(g + 1, 1 - b)
            si_start(g + 2, b)
            for j in range(2):
                copy_step(2 * g + j, j, True, True)
        return carry

    lax.fori_loop(1, 24, pair, 0)

    g = 48
    sg_wait(g, 0)
    sw_start(g, 0)
    si_wait(g + 1, 1)
    sw_wait(g - 1, 1)
    sg_start(g + 1, 1)
    copy_step(2 * g + 0, 0, True, True)
    copy_step(2 * g + 1, 1, True, True)
    g = 49
    sg_wait(g, 1)
    sw_start(g, 1)
    copy_step(2 * g + 0, 0, True, False)
    copy_step(2 * g + 1, 1, True, False)

    sw_wait(48, 0)
    sw_wait(49, 1)
    cw_wait(98, 0)
    cw_wait(99, 1)


def kernel(input_pos_tensors, table):
    B0, T = input_pos_tensors.shape
    V, D = table.shape
    B = B0 * T
    idx = input_pos_tensors.reshape(B).astype(jnp.int32)
    H = D // 2

    mesh = plsc.VectorSubcoreMesh(
        core_axis_name="c", subcore_axis_name="s",
        num_cores=_NC, num_subcores=_NS)
    run = pl.kernel(
        functools.partial(_body, H),
        out_type=jax.ShapeDtypeStruct((B, D), jnp.float32),
        mesh=mesh,
        scratch_types=[
            pltpu.VMEM_SHARED((V, D), jnp.float32),
            pltpu.VMEM((V, H), jnp.float32),
            pltpu.VMEM((_CS,), jnp.int32),
            pltpu.VMEM((_CS,), jnp.int32),
            pltpu.VMEM((_CS, D), jnp.float32),
            pltpu.VMEM((_CS, D), jnp.float32),
            pltpu.VMEM((_CC,), jnp.int32),
            pltpu.VMEM((_CC,), jnp.int32),
            pltpu.VMEM((_CC, H), jnp.float32),
            pltpu.VMEM((_CC, H), jnp.float32),
            pltpu.SemaphoreType.DMA,
            pltpu.SemaphoreType.DMA,
            pltpu.SemaphoreType.DMA,
            pltpu.SemaphoreType.DMA,
            pltpu.SemaphoreType.DMA,
            pltpu.SemaphoreType.DMA,
            pltpu.SemaphoreType.DMA,
            pltpu.SemaphoreType.DMA,
            pltpu.SemaphoreType.DMA,
            pltpu.SemaphoreType.DMA,
            pltpu.SemaphoreType.DMA,
        ],
        compiler_params=pltpu.CompilerParams(
            use_tc_tiling_on_sc=False, needs_layout_passes=False),
    )
    out = run(table, idx)
    return out.reshape(B0, T, D)


# speedup vs baseline: 2.3510x; 1.2417x over previous
import functools
import jax
import jax.numpy as jnp
from jax import lax
from jax.experimental import pallas as pl
from jax.experimental.pallas import tpu as pltpu
from jax.experimental.pallas import tpu_sc as plsc

_NC = 2
_NS = 16
_NW = _NC * _NS
_L = 16
_BB = 512


def _body(T, B0, H, ttab_hbm, idxt_hbm, out_hbm,
          ttile, idx0, idx1, stag0, stag1,
          sem_t, sem_i0, sem_i1, sem_w0, sem_w1):
    nblk = B0 // _BB
    nunits = T * nblk // (_NW // 2)
    wid = lax.axis_index("s") * _NC + lax.axis_index("c")
    ig = wid // 2
    h = wid % 2
    idxb = (idx0, idx1)
    stag = (stag0, stag1)
    sem_i = (sem_i0, sem_i1)
    sem_w = (sem_w0, sem_w1)

    pltpu.async_copy(ttab_hbm.at[pl.ds(h * H, H)], ttile, sem_t).wait()

    def unit_coords(u):
        g = ig + u * (_NW // 2)
        return g // nblk, g % nblk

    def idx_start(u, b):
        t, blk = unit_coords(u)
        pltpu.async_copy(
            idxt_hbm.at[t, pl.ds(blk * (_BB // _L), _BB // _L)],
            idxb[b], sem_i[b])

    def idx_wait(u, b):
        t, blk = unit_coords(u)
        pltpu.make_async_copy(
            idxt_hbm.at[t, pl.ds(blk * (_BB // _L), _BB // _L)],
            idxb[b], sem_i[b]).wait()

    def w_start(u, b):
        t, blk = unit_coords(u)
        pltpu.async_copy(
            stag[b], out_hbm.at[t, pl.ds(h * H, H), pl.ds(blk * _BB, _BB)],
            sem_w[b])

    def w_wait(u, b):
        t, blk = unit_coords(u)
        pltpu.make_async_copy(
            stag[b], out_hbm.at[t, pl.ds(h * H, H), pl.ds(blk * _BB, _BB)],
            sem_w[b]).wait()

    cols = [jnp.full((_L,), d, jnp.int32) for d in range(H)]

    def compute(b):
        @plsc.parallel_loop(0, _BB // _L, unroll=2)
        def group(g):
            iv = idxb[b][g]
            for d in range(H):
                vals = plsc.load_gather(ttile, [cols[d], iv])
                stag[b][d, pl.ds(g * _L, _L)] = vals

    for b in range(2):
        idx_start(b, b)

    def pair(p, carry):
        for b in range(2):
            u = 2 * p + b
            idx_wait(u, b)
            compute(b)
            w_start(u, b)
            w_wait(u, b)
            idx_start(u + 2, b)
        return carry

    npairs = nunits // 2
    lax.fori_loop(0, npairs - 1, pair, 0)

    for b in range(2):
        u = (npairs - 1) * 2 + b
        idx_wait(u, b)
        compute(b)
        w_start(u, b)
    for b in range(2):
        u = (npairs - 1) * 2 + b
        w_wait(u, b)


def kernel(input_pos_tensors, table):
    B0, T = input_pos_tensors.shape
    V, D = table.shape
    H = D // 2
    ttab = table.T
    idxt = input_pos_tensors.astype(jnp.int32).T.reshape(T, B0 // _L, _L)

    mesh = plsc.VectorSubcoreMesh(
        core_axis_name="c", subcore_axis_name="s",
        num_cores=_NC, num_subcores=_NS)
    run = pl.kernel(
        functools.partial(_body, T, B0, H),
        out_type=jax.ShapeDtypeStruct((T, D, B0), jnp.float32),
        mesh=mesh,
        scratch_types=[
            pltpu.VMEM((H, V), jnp.float32),
            pltpu.VMEM((_BB // _L, _L), jnp.int32),
            pltpu.VMEM((_BB // _L, _L), jnp.int32),
            pltpu.VMEM((H, _BB), jnp.float32),
            pltpu.VMEM((H, _BB), jnp.float32),
            pltpu.SemaphoreType.DMA,
            pltpu.SemaphoreType.DMA,
            pltpu.SemaphoreType.DMA,
            pltpu.SemaphoreType.DMA,
            pltpu.SemaphoreType.DMA,
        ],
        compiler_params=pltpu.CompilerParams(
            use_tc_tiling_on_sc=False, needs_layout_passes=False),
    )
    out = run(ttab, idxt)
    return jnp.transpose(out, (2, 0, 1))
